# Initial kernel scaffold; baseline (speedup 1.0000x reference)
import jax
import jax.numpy as jnp
from jax import lax
from jax.experimental import pallas as pl
from jax.experimental.pallas import tpu as pltpu
from jax.experimental.pallas import tpu_sc as plsc

N_BITS = 14
N_SLOTS = 2 ** N_BITS
NTOK = 4096
D = 1024
BLK = 512
GRID = NTOK // BLK

NC, NS, LANES = 2, 16, 16
NW = NC * NS
TPW = NTOK // NW
CH = 32
NCHUNK = TPW // CH


def _hash_body(wk_ref, hp_ref, bk_ref, h_ref, idx_ref, wh_s, bh_s):
    @pl.when(pl.program_id(0) == 0)
    def _fold():
        wh_s[...] = jnp.dot(wk_ref[...], hp_ref[...],
                            preferred_element_type=jnp.float32)
        bh_s[...] = jnp.dot(bk_ref[...], hp_ref[...],
                            preferred_element_type=jnp.float32)
    t = jnp.dot(h_ref[...], wh_s[...],
                preferred_element_type=jnp.float32) + bh_s[...]
    lane = lax.broadcasted_iota(jnp.int32, (BLK, 128), 1)
    powers = jnp.where(lane < N_BITS, jnp.left_shift(1, lane), 0)
    bits = (t > 0.0).astype(jnp.int32)
    idx_ref[...] = jnp.sum(bits * powers, axis=1, keepdims=True)


def _hash_call(h2d, w_k, hp_pad, bk2d):
    return pl.pallas_call(
        _hash_body,
        grid=(GRID,),
        in_specs=[
            pl.BlockSpec((1024, 1024), lambda i: (0, 0)),
            pl.BlockSpec((1024, 128), lambda i: (0, 0)),
            pl.BlockSpec((1, 1024), lambda i: (0, 0)),
            pl.BlockSpec((BLK, 1024), lambda i: (i, 0)),
        ],
        out_specs=pl.BlockSpec((BLK, 1), lambda i: (i, 0)),
        out_shape=jax.ShapeDtypeStruct((NTOK, 1), jnp.int32),
        scratch_shapes=[
            pltpu.VMEM((1024, 128), jnp.float32),
            pltpu.VMEM((1, 128), jnp.float32),
        ],
    )(w_k, hp_pad, bk2d, h2d)


def _sc_gather_body(vals_hbm, counts_hbm, idx_hbm, pred_hbm, conf_hbm,
                    idx_v, counts_v, rows0, rows1, conf_v, sem0, sem1):
    wid = lax.axis_index("s") * NC + lax.axis_index("c")
    base = wid * TPW
    pltpu.sync_copy(idx_hbm.at[pl.ds(base, TPW)], idx_v)
    pltpu.sync_copy(counts_hbm, counts_v)
    for i in range(TPW // LANES):
        iv = idx_v[pl.ds(i * LANES, LANES)]
        c = plsc.load_gather(counts_v, [iv])
        conf_v[pl.ds(i * LANES, LANES)] = c / (c + 1.0)
    pltpu.sync_copy(conf_v, conf_hbm.at[pl.ds(base, TPW)])

    rows = (rows0, rows1)
    sems = (sem0, sem1)
    cps = []
    for k in range(NCHUNK):
        b = k % 2
        if k >= 2:
            cps[k - 2].wait()
            pltpu.sync_copy(rows[b],
                            pred_hbm.at[pl.ds(base + (k - 2) * CH, CH)])
        cp = pltpu.async_copy(vals_hbm.at[idx_v.at[pl.ds(k * CH, CH)]],
                              rows[b], sems[b])
        cps.append(cp)
    for k in range(max(NCHUNK - 2, 0), NCHUNK):
        cps[k].wait()
        pltpu.sync_copy(rows[k % 2], pred_hbm.at[pl.ds(base + k * CH, CH)])


def _sc_gather_call(mem_vals, mem_counts, idx1d):
    mesh = plsc.VectorSubcoreMesh(core_axis_name="c", subcore_axis_name="s",
                                  num_cores=NC, num_subcores=NS)
    fn = pl.kernel(
        _sc_gather_body,
        out_type=[
            jax.ShapeDtypeStruct((NTOK, D), jnp.float32),
            jax.ShapeDtypeStruct((NTOK,), jnp.float32),
        ],
        mesh=mesh,
        scratch_types=[
            pltpu.VMEM((TPW,), jnp.int32),
            pltpu.VMEM((N_SLOTS,), jnp.float32),
            pltpu.VMEM((CH, D), jnp.float32),
            pltpu.VMEM((CH, D), jnp.float32),
            pltpu.VMEM((TPW,), jnp.float32),
            pltpu.SemaphoreType.DMA,
            pltpu.SemaphoreType.DMA,
        ],
    )
    return fn(mem_vals, mem_counts, idx1d)


def _ln(x, g, b):
    mu = jnp.mean(x, axis=1, keepdims=True)
    xc = x - mu
    var = jnp.mean(xc * xc, axis=1, keepdims=True)
    return xc * lax.rsqrt(var + 1e-5) * g + b


def _gate_body(h_ref, pred_ref, conf_ref, w1h_ref, w1p_ref, w1c_ref, ba1_ref,
               wa2_ref, ba2_ref, gin_ref, btin_ref, gp_ref, btp_ref,
               y_ref, asum_ref, csum_ref):
    x = h_ref[...]
    p = pred_ref[...]
    cf = conf_ref[...]
    ln_h = _ln(x, gin_ref[...], btin_ref[...])
    ln_p = _ln(p, gp_ref[...], btp_ref[...])
    z = (jnp.dot(ln_h, w1h_ref[...], preferred_element_type=jnp.float32)
         + jnp.dot(ln_p, w1p_ref[...], preferred_element_type=jnp.float32)
         + cf * w1c_ref[...] + ba1_ref[...])
    hid = z * jax.nn.sigmoid(z)
    s = jnp.sum(hid * wa2_ref[...], axis=1, keepdims=True) + ba2_ref[...]
    alpha = jax.nn.sigmoid(s)
    y_ref[...] = x + alpha * p
    asum_ref[0, 0] = jnp.sum(alpha)
    csum_ref[0, 0] = jnp.sum(cf)


def _gate_call(h2d, pred, conf2d, w1h, w1p, w1c, ba1, wa2, ba2,
               gin, btin, gp, btp):
    def full(r, c):
        return pl.BlockSpec((r, c), lambda i: (0, 0))
    return pl.pallas_call(
        _gate_body,
        grid=(GRID,),
        in_specs=[
            pl.BlockSpec((BLK, D), lambda i: (i, 0)),
            pl.BlockSpec((BLK, D), lambda i: (i, 0)),
            pl.BlockSpec((BLK, 1), lambda i: (i, 0)),
            full(1024, 1024),
            full(1024, 1024),
            full(1, 1024),
            full(1, 1024),
            full(1, 1024),
            full(1, 1),
            full(1, 1024),
            full(1, 1024),
            full(1, 1024),
            full(1, 1024),
        ],
        out_specs=[
            pl.BlockSpec((BLK, D), lambda i: (i, 0)),
            pl.BlockSpec((1, 1), lambda i: (i, 0),
                         memory_space=pltpu.SMEM),
            pl.BlockSpec((1, 1), lambda i: (i, 0),
                         memory_space=pltpu.SMEM),
        ],
        out_shape=[
            jax.ShapeDtypeStruct((NTOK, D), jnp.float32),
            jax.ShapeDtypeStruct((GRID, 1), jnp.float32),
            jax.ShapeDtypeStruct((GRID, 1), jnp.float32),
        ],
    )(h2d, pred, conf2d, w1h, w1p, w1c, ba1, wa2, ba2, gin, btin, gp, btp)


def kernel(h_in, W_k, b_k, W_a1, b_a1, W_a2, b_a2, g_in, bt_in,
           g_pred, bt_pred, hash_proj, mem_vals, mem_counts):
    Bx, Lx, Dx = h_in.shape
    h2d = h_in.reshape(Bx * Lx, Dx)

    hp_pad = jnp.zeros((Dx, 128), jnp.float32).at[:, :N_BITS].set(hash_proj)
    idx = _hash_call(h2d, W_k, hp_pad, b_k.reshape(1, Dx))

    pred, conf = _sc_gather_call(mem_vals, mem_counts, idx.reshape(NTOK))

    w1h = W_a1[:Dx]
    w1p = W_a1[Dx:2 * Dx]
    w1c = W_a1[2 * Dx:2 * Dx + 1]
    y, asum, csum = _gate_call(
        h2d, pred, conf.reshape(NTOK, 1), w1h, w1p, w1c,
        b_a1.reshape(1, Dx), W_a2.reshape(1, Dx), b_a2.reshape(1, 1),
        g_in.reshape(1, Dx), bt_in.reshape(1, Dx),
        g_pred.reshape(1, Dx), bt_pred.reshape(1, Dx),
    )
    y_out = y.reshape(Bx, Lx, Dx)
    a_mean = jnp.sum(asum) / jnp.float32(NTOK)
    c_mean = jnp.sum(csum) / jnp.float32(NTOK)
    return (y_out, a_mean, c_mean)


# speedup vs baseline: 1.9104x; 1.9104x over previous
import jax
import jax.numpy as jnp
from jax import lax
from jax.experimental import pallas as pl
from jax.experimental.pallas import tpu as pltpu
from jax.experimental.pallas import tpu_sc as plsc

N_BITS = 14
N_SLOTS = 2 ** N_BITS
NTOK = 4096
D = 1024
BLK = 512
GRID = NTOK // BLK

NC, NS, LANES = 2, 16, 16
NW = NC * NS
TPW = NTOK // NW
CH = 32
NCHUNK = TPW // CH


def _hash_body(wk_ref, hp_ref, bk_ref, h_ref, idx_ref, wh_s, bh_s):
    @pl.when(pl.program_id(0) == 0)
    def _fold():
        wh_s[...] = jnp.dot(wk_ref[...], hp_ref[...],
                            preferred_element_type=jnp.float32)
        bh_s[...] = jnp.dot(bk_ref[...], hp_ref[...],
                            preferred_element_type=jnp.float32)
    t = jnp.dot(h_ref[...], wh_s[...],
                preferred_element_type=jnp.float32) + bh_s[...]
    lane = lax.broadcasted_iota(jnp.int32, (BLK, 128), 1)
    powers = jnp.where(lane < N_BITS, jnp.left_shift(1, lane), 0)
    bits = (t > 0.0).astype(jnp.int32)
    idx_ref[...] = jnp.sum(bits * powers, axis=1, keepdims=True)


def _hash_call(h2d, w_k, hp_pad, bk2d):
    return pl.pallas_call(
        _hash_body,
        grid=(GRID,),
        in_specs=[
            pl.BlockSpec((1024, 1024), lambda i: (0, 0)),
            pl.BlockSpec((1024, 128), lambda i: (0, 0)),
            pl.BlockSpec((1, 1024), lambda i: (0, 0)),
            pl.BlockSpec((BLK, 1024), lambda i: (i, 0)),
        ],
        out_specs=pl.BlockSpec((BLK, 1), lambda i: (i, 0)),
        out_shape=jax.ShapeDtypeStruct((NTOK, 1), jnp.int32),
        scratch_shapes=[
            pltpu.VMEM((1024, 128), jnp.float32),
            pltpu.VMEM((1, 128), jnp.float32),
        ],
    )(w_k, hp_pad, bk2d, h2d)


def _sc_gather_body(vals_hbm, counts_hbm, idx_hbm, pred_hbm, conf_hbm,
                    idx_v, counts_v, rows0, rows1, conf_v, sem0, sem1):
    wid = lax.axis_index("s") * NC + lax.axis_index("c")
    base = wid * TPW
    pltpu.sync_copy(idx_hbm.at[pl.ds(base, TPW)], idx_v)
    pltpu.sync_copy(counts_hbm, counts_v)
    for i in range(TPW // LANES):
        iv = idx_v[pl.ds(i * LANES, LANES)]
        c = plsc.load_gather(counts_v, [iv])
        conf_v[pl.ds(i * LANES, LANES)] = c / (c + 1.0)
    pltpu.sync_copy(conf_v, conf_hbm.at[pl.ds(base, TPW)])

    rows = (rows0, rows1)
    sems = (sem0, sem1)
    cps = []
    for k in range(NCHUNK):
        b = k % 2
        if k >= 2:
            cps[k - 2].wait()
            pltpu.sync_copy(rows[b],
                            pred_hbm.at[pl.ds(base + (k - 2) * CH, CH)])
        cp = pltpu.async_copy(vals_hbm.at[idx_v.at[pl.ds(k * CH, CH)]],
                              rows[b], sems[b])
        cps.append(cp)
    for k in range(max(NCHUNK - 2, 0), NCHUNK):
        cps[k].wait()
        pltpu.sync_copy(rows[k % 2], pred_hbm.at[pl.ds(base + k * CH, CH)])


def _sc_gather_call(mem_vals, mem_counts, idx1d):
    mesh = plsc.VectorSubcoreMesh(core_axis_name="c", subcore_axis_name="s",
                                  num_cores=NC, num_subcores=NS)
    fn = pl.kernel(
        _sc_gather_body,
        compiler_params=pltpu.CompilerParams(needs_layout_passes=False),
        out_type=[
            jax.ShapeDtypeStruct((NTOK, D), jnp.float32),
            jax.ShapeDtypeStruct((NTOK,), jnp.float32),
        ],
        mesh=mesh,
        scratch_types=[
            pltpu.VMEM((TPW,), jnp.int32),
            pltpu.VMEM((N_SLOTS,), jnp.float32),
            pltpu.VMEM((CH, D), jnp.float32),
            pltpu.VMEM((CH, D), jnp.float32),
            pltpu.VMEM((TPW,), jnp.float32),
            pltpu.SemaphoreType.DMA,
            pltpu.SemaphoreType.DMA,
        ],
    )
    return fn(mem_vals, mem_counts, idx1d)


def _ln(x, g, b):
    mu = jnp.mean(x, axis=1, keepdims=True)
    xc = x - mu
    var = jnp.mean(xc * xc, axis=1, keepdims=True)
    return xc * lax.rsqrt(var + 1e-5) * g + b


def _gate_body(h_ref, pred_ref, conf_ref, w1h_ref, w1p_ref, w1c_ref, ba1_ref,
               wa2_ref, ba2_ref, gin_ref, btin_ref, gp_ref, btp_ref,
               y_ref, asum_ref, csum_ref):
    x = h_ref[...]
    p = pred_ref[...]
    cf = conf_ref[...]
    ln_h = _ln(x, gin_ref[...], btin_ref[...])
    ln_p = _ln(p, gp_ref[...], btp_ref[...])
    z = (jnp.dot(ln_h, w1h_ref[...], preferred_element_type=jnp.float32)
         + jnp.dot(ln_p, w1p_ref[...], preferred_element_type=jnp.float32)
         + cf * w1c_ref[...] + ba1_ref[...])
    hid = z * jax.nn.sigmoid(z)
    s = jnp.sum(hid * wa2_ref[...], axis=1, keepdims=True) + ba2_ref[...]
    alpha = jax.nn.sigmoid(s)
    y_ref[...] = x + alpha * p

    @pl.when(pl.program_id(0) == 0)
    def _init():
        asum_ref[0, 0] = 0.0
        csum_ref[0, 0] = 0.0
    asum_ref[0, 0] += jnp.sum(alpha)
    csum_ref[0, 0] += jnp.sum(cf)


def _gate_call(h2d, pred, conf2d, w1h, w1p, w1c, ba1, wa2, ba2,
               gin, btin, gp, btp):
    def full(r, c):
        return pl.BlockSpec((r, c), lambda i: (0, 0))
    return pl.pallas_call(
        _gate_body,
        grid=(GRID,),
        in_specs=[
            pl.BlockSpec((BLK, D), lambda i: (i, 0)),
            pl.BlockSpec((BLK, D), lambda i: (i, 0)),
            pl.BlockSpec((BLK, 1), lambda i: (i, 0)),
            full(1024, 1024),
            full(1024, 1024),
            full(1, 1024),
            full(1, 1024),
            full(1, 1024),
            full(1, 1),
            full(1, 1024),
            full(1, 1024),
            full(1, 1024),
            full(1, 1024),
        ],
        out_specs=[
            pl.BlockSpec((BLK, D), lambda i: (i, 0)),
            pl.BlockSpec((1, 1), lambda i: (0, 0),
                         memory_space=pltpu.SMEM),
            pl.BlockSpec((1, 1), lambda i: (0, 0),
                         memory_space=pltpu.SMEM),
        ],
        out_shape=[
            jax.ShapeDtypeStruct((NTOK, D), jnp.float32),
            jax.ShapeDtypeStruct((1, 1), jnp.float32),
            jax.ShapeDtypeStruct((1, 1), jnp.float32),
        ],
    )(h2d, pred, conf2d, w1h, w1p, w1c, ba1, wa2, ba2, gin, btin, gp, btp)


def kernel(h_in, W_k, b_k, W_a1, b_a1, W_a2, b_a2, g_in, bt_in,
           g_pred, bt_pred, hash_proj, mem_vals, mem_counts):
    Bx, Lx, Dx = h_in.shape
    h2d = h_in.reshape(Bx * Lx, Dx)

    hp_pad = jnp.zeros((Dx, 128), jnp.float32).at[:, :N_BITS].set(hash_proj)
    idx = _hash_call(h2d, W_k, hp_pad, b_k.reshape(1, Dx))

    pred, conf = _sc_gather_call(mem_vals, mem_counts, idx.reshape(NTOK))

    w1h = W_a1[:Dx]
    w1p = W_a1[Dx:2 * Dx]
    w1c = W_a1[2 * Dx:2 * Dx + 1]
    y, asum, csum = _gate_call(
        h2d, pred, conf.reshape(NTOK, 1), w1h, w1p, w1c,
        b_a1.reshape(1, Dx), W_a2.reshape(1, Dx), b_a2.reshape(1, 1),
        g_in.reshape(1, Dx), bt_in.reshape(1, Dx),
        g_pred.reshape(1, Dx), bt_pred.reshape(1, Dx),
    )
    y_out = y.reshape(Bx, Lx, Dx)
    a_mean = jnp.sum(asum) / jnp.float32(NTOK)
    c_mean = jnp.sum(csum) / jnp.float32(NTOK)
    return (y_out, a_mean, c_mean)


# speedup vs baseline: 1.9523x; 1.0219x over previous
import jax
import jax.numpy as jnp
from jax import lax
from jax.experimental import pallas as pl
from jax.experimental.pallas import tpu as pltpu
from jax.experimental.pallas import tpu_sc as plsc

N_BITS = 14
N_SLOTS = 2 ** N_BITS
NTOK = 4096
D = 1024
BLK = 512
GRID = NTOK // BLK

NC, NS, LANES = 2, 16, 16
NW = NC * NS
TPW = NTOK // NW
CH = 32
NCHUNK = TPW // CH


def _hash_body(wk_ref, hp_ref, bk_ref, h_ref, idx_ref, wh_s, bh_s):
    @pl.when(pl.program_id(0) == 0)
    def _fold():
        wh_s[...] = jnp.dot(wk_ref[...], hp_ref[...],
                            preferred_element_type=jnp.float32).astype(
                                jnp.bfloat16)
        bh_s[...] = jnp.dot(bk_ref[...], hp_ref[...],
                            preferred_element_type=jnp.float32)
    t = jnp.dot(h_ref[...].astype(jnp.bfloat16), wh_s[...],
                preferred_element_type=jnp.float32) + bh_s[...]
    lane = lax.broadcasted_iota(jnp.int32, (BLK, 128), 1)
    powers = jnp.where(lane < N_BITS, jnp.left_shift(1, lane), 0)
    bits = (t > 0.0).astype(jnp.int32)
    idx_ref[...] = jnp.sum(bits * powers, axis=1, keepdims=True)


def _hash_call(h2d, w_k, hp_pad, bk2d):
    return pl.pallas_call(
        _hash_body,
        grid=(GRID,),
        in_specs=[
            pl.BlockSpec((1024, 1024), lambda i: (0, 0)),
            pl.BlockSpec((1024, 128), lambda i: (0, 0)),
            pl.BlockSpec((1, 1024), lambda i: (0, 0)),
            pl.BlockSpec((BLK, 1024), lambda i: (i, 0)),
        ],
        out_specs=pl.BlockSpec((BLK, 1), lambda i: (i, 0)),
        out_shape=jax.ShapeDtypeStruct((NTOK, 1), jnp.int32),
        scratch_shapes=[
            pltpu.VMEM((1024, 128), jnp.bfloat16),
            pltpu.VMEM((1, 128), jnp.float32),
        ],
    )(w_k, hp_pad, bk2d, h2d)


def _sc_gather_body(vals_hbm, counts_hbm, idx_hbm, pred_hbm, conf_hbm,
                    idx_v, counts_v, rows0, rows1, conf_v, sem0, sem1):
    wid = lax.axis_index("s") * NC + lax.axis_index("c")
    base = wid * TPW
    pltpu.sync_copy(idx_hbm.at[pl.ds(base, TPW)], idx_v)
    pltpu.sync_copy(counts_hbm, counts_v)
    for i in range(TPW // LANES):
        iv = idx_v[pl.ds(i * LANES, LANES)]
        c = plsc.load_gather(counts_v, [iv])
        conf_v[pl.ds(i * LANES, LANES)] = c / (c + 1.0)
    pltpu.sync_copy(conf_v, conf_hbm.at[pl.ds(base, TPW)])

    rows = (rows0, rows1)
    sems = (sem0, sem1)
    cps = []
    for k in range(NCHUNK):
        b = k % 2
        if k >= 2:
            cps[k - 2].wait()
            pltpu.sync_copy(rows[b],
                            pred_hbm.at[pl.ds(base + (k - 2) * CH, CH)])
        cp = pltpu.async_copy(vals_hbm.at[idx_v.at[pl.ds(k * CH, CH)]],
                              rows[b], sems[b])
        cps.append(cp)
    for k in range(max(NCHUNK - 2, 0), NCHUNK):
        cps[k].wait()
        pltpu.sync_copy(rows[k % 2], pred_hbm.at[pl.ds(base + k * CH, CH)])


def _sc_gather_call(mem_vals, mem_counts, idx1d):
    mesh = plsc.VectorSubcoreMesh(core_axis_name="c", subcore_axis_name="s",
                                  num_cores=NC, num_subcores=NS)
    fn = pl.kernel(
        _sc_gather_body,
        compiler_params=pltpu.CompilerParams(needs_layout_passes=False),
        out_type=[
            jax.ShapeDtypeStruct((NTOK, D), jnp.float32),
            jax.ShapeDtypeStruct((NTOK,), jnp.float32),
        ],
        mesh=mesh,
        scratch_types=[
            pltpu.VMEM((TPW,), jnp.int32),
            pltpu.VMEM((N_SLOTS,), jnp.float32),
            pltpu.VMEM((CH, D), jnp.float32),
            pltpu.VMEM((CH, D), jnp.float32),
            pltpu.VMEM((TPW,), jnp.float32),
            pltpu.SemaphoreType.DMA,
            pltpu.SemaphoreType.DMA,
        ],
    )
    return fn(mem_vals, mem_counts, idx1d)


def _ln(x, g, b):
    mu = jnp.mean(x, axis=1, keepdims=True)
    xc = x - mu
    var = jnp.mean(xc * xc, axis=1, keepdims=True)
    return xc * lax.rsqrt(var + 1e-5) * g + b


def _gate_body(h_ref, pred_ref, conf_ref, w1h_ref, w1p_ref, w1c_ref, ba1_ref,
               wa2_ref, ba2_ref, gin_ref, btin_ref, gp_ref, btp_ref,
               y_ref, asum_ref, csum_ref):
    x = h_ref[...]
    p = pred_ref[...]
    cf = conf_ref[...]
    ln_h = _ln(x, gin_ref[...], btin_ref[...])
    ln_p = _ln(p, gp_ref[...], btp_ref[...])
    z = (jnp.dot(ln_h.astype(jnp.bfloat16), w1h_ref[...],
                 preferred_element_type=jnp.float32)
         + jnp.dot(ln_p.astype(jnp.bfloat16), w1p_ref[...],
                   preferred_element_type=jnp.float32)
         + cf * w1c_ref[...] + ba1_ref[...])
    hid = z * jax.nn.sigmoid(z)
    s = jnp.sum(hid * wa2_ref[...], axis=1, keepdims=True) + ba2_ref[...]
    alpha = jax.nn.sigmoid(s)
    y_ref[...] = x + alpha * p

    @pl.when(pl.program_id(0) == 0)
    def _init():
        asum_ref[0, 0] = 0.0
        csum_ref[0, 0] = 0.0
    asum_ref[0, 0] += jnp.sum(alpha)
    csum_ref[0, 0] += jnp.sum(cf)


def _gate_call(h2d, pred, conf2d, w1h, w1p, w1c, ba1, wa2, ba2,
               gin, btin, gp, btp):
    def full(r, c):
        return pl.BlockSpec((r, c), lambda i: (0, 0))
    return pl.pallas_call(
        _gate_body,
        grid=(GRID,),
        in_specs=[
            pl.BlockSpec((BLK, D), lambda i: (i, 0)),
            pl.BlockSpec((BLK, D), lambda i: (i, 0)),
            pl.BlockSpec((BLK, 1), lambda i: (i, 0)),
            full(1024, 1024),
            full(1024, 1024),
            full(1, 1024),
            full(1, 1024),
            full(1, 1024),
            full(1, 1),
            full(1, 1024),
            full(1, 1024),
            full(1, 1024),
            full(1, 1024),
        ],
        out_specs=[
            pl.BlockSpec((BLK, D), lambda i: (i, 0)),
            pl.BlockSpec((1, 1), lambda i: (0, 0),
                         memory_space=pltpu.SMEM),
            pl.BlockSpec((1, 1), lambda i: (0, 0),
                         memory_space=pltpu.SMEM),
        ],
        out_shape=[
            jax.ShapeDtypeStruct((NTOK, D), jnp.float32),
            jax.ShapeDtypeStruct((1, 1), jnp.float32),
            jax.ShapeDtypeStruct((1, 1), jnp.float32),
        ],
    )(h2d, pred, conf2d, w1h, w1p, w1c, ba1, wa2, ba2, gin, btin, gp, btp)


def kernel(h_in, W_k, b_k, W_a1, b_a1, W_a2, b_a2, g_in, bt_in,
           g_pred, bt_pred, hash_proj, mem_vals, mem_counts):
    Bx, Lx, Dx = h_in.shape
    h2d = h_in.reshape(Bx * Lx, Dx)

    hp_pad = jnp.zeros((Dx, 128), jnp.float32).at[:, :N_BITS].set(hash_proj)
    idx = _hash_call(h2d, W_k, hp_pad, b_k.reshape(1, Dx))

    pred, conf = _sc_gather_call(mem_vals, mem_counts, idx.reshape(NTOK))

    w1h = W_a1[:Dx].astype(jnp.bfloat16)
    w1p = W_a1[Dx:2 * Dx].astype(jnp.bfloat16)
    w1c = W_a1[2 * Dx:2 * Dx + 1]
    y, asum, csum = _gate_call(
        h2d, pred, conf.reshape(NTOK, 1), w1h, w1p, w1c,
        b_a1.reshape(1, Dx), W_a2.reshape(1, Dx), b_a2.reshape(1, 1),
        g_in.reshape(1, Dx), bt_in.reshape(1, Dx),
        g_pred.reshape(1, Dx), bt_pred.reshape(1, Dx),
    )
    y_out = y.reshape(Bx, Lx, Dx)
    a_mean = jnp.sum(asum) / jnp.float32(NTOK)
    c_mean = jnp.sum(csum) / jnp.float32(NTOK)
    return (y_out, a_mean, c_mean)


# speedup vs baseline: 2.0153x; 1.0323x over previous
import jax
import jax.numpy as jnp
from jax import lax
from jax.experimental import pallas as pl
from jax.experimental.pallas import tpu as pltpu
from jax.experimental.pallas import tpu_sc as plsc

N_BITS = 14
N_SLOTS = 2 ** N_BITS
NTOK = 4096
D = 1024
BLK = 512
GRID = NTOK // BLK

NC, NS, LANES = 2, 16, 16
NW = NC * NS
TPW = NTOK // NW
CH = 32
NCHUNK = TPW // CH


def _hash_body(wk_ref, hp_ref, bk_ref, h_ref, idx_ref, wh_s, bh_s):
    @pl.when(pl.program_id(0) == 0)
    def _fold():
        wh_s[...] = jnp.dot(wk_ref[...], hp_ref[...],
                            preferred_element_type=jnp.float32).astype(
                                jnp.bfloat16)
        bh_s[...] = jnp.dot(bk_ref[...], hp_ref[...],
                            preferred_element_type=jnp.float32)
    t = jnp.dot(h_ref[...].astype(jnp.bfloat16), wh_s[...],
                preferred_element_type=jnp.float32) + bh_s[...]
    lane = lax.broadcasted_iota(jnp.int32, (BLK, N_BITS), 1)
    powers = jnp.left_shift(1, lane)
    bits = (t > 0.0).astype(jnp.int32)
    idx_ref[...] = jnp.sum(bits * powers, axis=1, keepdims=True)


def _hash_call(h2d, w_k, hp, bk2d):
    return pl.pallas_call(
        _hash_body,
        grid=(GRID,),
        in_specs=[
            pl.BlockSpec((1024, 1024), lambda i: (0, 0)),
            pl.BlockSpec((1024, N_BITS), lambda i: (0, 0)),
            pl.BlockSpec((1, 1024), lambda i: (0, 0)),
            pl.BlockSpec((BLK, 1024), lambda i: (i, 0)),
        ],
        out_specs=pl.BlockSpec((BLK, 1), lambda i: (i, 0)),
        out_shape=jax.ShapeDtypeStruct((NTOK, 1), jnp.int32),
        scratch_shapes=[
            pltpu.VMEM((1024, N_BITS), jnp.bfloat16),
            pltpu.VMEM((1, N_BITS), jnp.float32),
        ],
    )(w_k, hp, bk2d, h2d)


def _sc_gather_body(vals_hbm, counts_hbm, idx_hbm, pred_hbm, conf_hbm,
                    idx_v, counts_v, rows0, rows1, conf_v, sem0, sem1):
    wid = lax.axis_index("s") * NC + lax.axis_index("c")
    base = wid * TPW
    pltpu.sync_copy(idx_hbm.at[pl.ds(base, TPW)], idx_v)
    pltpu.sync_copy(counts_hbm, counts_v)
    for i in range(TPW // LANES):
        iv = idx_v[pl.ds(i * LANES, LANES)]
        c = plsc.load_gather(counts_v, [iv])
        conf_v[pl.ds(i * LANES, LANES)] = c / (c + 1.0)
    pltpu.sync_copy(conf_v, conf_hbm.at[pl.ds(base, TPW)])

    rows = (rows0, rows1)
    sems = (sem0, sem1)
    cps = []
    for k in range(NCHUNK):
        b = k % 2
        if k >= 2:
            cps[k - 2].wait()
            pltpu.sync_copy(rows[b],
                            pred_hbm.at[pl.ds(base + (k - 2) * CH, CH)])
        cp = pltpu.async_copy(vals_hbm.at[idx_v.at[pl.ds(k * CH, CH)]],
                              rows[b], sems[b])
        cps.append(cp)
    for k in range(max(NCHUNK - 2, 0), NCHUNK):
        cps[k].wait()
        pltpu.sync_copy(rows[k % 2], pred_hbm.at[pl.ds(base + k * CH, CH)])


def _sc_gather_call(mem_vals, mem_counts, idx1d):
    mesh = plsc.VectorSubcoreMesh(core_axis_name="c", subcore_axis_name="s",
                                  num_cores=NC, num_subcores=NS)
    fn = pl.kernel(
        _sc_gather_body,
        compiler_params=pltpu.CompilerParams(needs_layout_passes=False),
        out_type=[
            jax.ShapeDtypeStruct((NTOK, D), jnp.float32),
            jax.ShapeDtypeStruct((NTOK,), jnp.float32),
        ],
        mesh=mesh,
        scratch_types=[
            pltpu.VMEM((TPW,), jnp.int32),
            pltpu.VMEM((N_SLOTS,), jnp.float32),
            pltpu.VMEM((CH, D), jnp.float32),
            pltpu.VMEM((CH, D), jnp.float32),
            pltpu.VMEM((TPW,), jnp.float32),
            pltpu.SemaphoreType.DMA,
            pltpu.SemaphoreType.DMA,
        ],
    )
    return fn(mem_vals, mem_counts, idx1d)


def _ln(x, g, b):
    mu = jnp.mean(x, axis=1, keepdims=True)
    xc = x - mu
    var = jnp.mean(xc * xc, axis=1, keepdims=True)
    return xc * lax.rsqrt(var + 1e-5) * g + b


def _gate_body(h_ref, pred_ref, conf_ref, w1h_ref, w1p_ref, w1c_ref, ba1_ref,
               wa2_ref, ba2_ref, gin_ref, btin_ref, gp_ref, btp_ref,
               y_ref, asum_ref, csum_ref, w1hb_s, w1pb_s):
    @pl.when(pl.program_id(0) == 0)
    def _cast():
        w1hb_s[...] = w1h_ref[...].astype(jnp.bfloat16)
        w1pb_s[...] = w1p_ref[...].astype(jnp.bfloat16)
    x = h_ref[...]
    p = pred_ref[...]
    cf = conf_ref[...]
    ln_h = _ln(x, gin_ref[...], btin_ref[...])
    ln_p = _ln(p, gp_ref[...], btp_ref[...])
    z = (jnp.dot(ln_h.astype(jnp.bfloat16), w1hb_s[...],
                 preferred_element_type=jnp.float32)
         + jnp.dot(ln_p.astype(jnp.bfloat16), w1pb_s[...],
                   preferred_element_type=jnp.float32)
         + cf * w1c_ref[0:1, :] + ba1_ref[...])
    hid = z * jax.nn.sigmoid(z)
    s = jnp.sum(hid * wa2_ref[...], axis=1, keepdims=True) + ba2_ref[...]
    alpha = jax.nn.sigmoid(s)
    y_ref[...] = x + alpha * p

    @pl.when(pl.program_id(0) == 0)
    def _init():
        asum_ref[0, 0] = 0.0
        csum_ref[0, 0] = 0.0
    asum_ref[0, 0] += jnp.sum(alpha)
    csum_ref[0, 0] += jnp.sum(cf)


def _gate_call(h2d, pred, conf2d, w_a1, ba1, wa2, ba2,
               gin, btin, gp, btp):
    def full(r, c):
        return pl.BlockSpec((r, c), lambda i: (0, 0))
    return pl.pallas_call(
        _gate_body,
        grid=(GRID,),
        in_specs=[
            pl.BlockSpec((BLK, D), lambda i: (i, 0)),
            pl.BlockSpec((BLK, D), lambda i: (i, 0)),
            pl.BlockSpec((BLK, 1), lambda i: (i, 0)),
            pl.BlockSpec((1024, 1024), lambda i: (0, 0)),
            pl.BlockSpec((1024, 1024), lambda i: (1, 0)),
            pl.BlockSpec((8, 1024), lambda i: (256, 0)),
            full(1, 1024),
            full(1, 1024),
            full(1, 1),
            full(1, 1024),
            full(1, 1024),
            full(1, 1024),
            full(1, 1024),
        ],
        out_specs=[
            pl.BlockSpec((BLK, D), lambda i: (i, 0)),
            pl.BlockSpec((1, 1), lambda i: (0, 0),
                         memory_space=pltpu.SMEM),
            pl.BlockSpec((1, 1), lambda i: (0, 0),
                         memory_space=pltpu.SMEM),
        ],
        out_shape=[
            jax.ShapeDtypeStruct((NTOK, D), jnp.float32),
            jax.ShapeDtypeStruct((1, 1), jnp.float32),
            jax.ShapeDtypeStruct((1, 1), jnp.float32),
        ],
        scratch_shapes=[
            pltpu.VMEM((1024, 1024), jnp.bfloat16),
            pltpu.VMEM((1024, 1024), jnp.bfloat16),
        ],
    )(h2d, pred, conf2d, w_a1, w_a1, w_a1, ba1, wa2, ba2, gin, btin, gp, btp)


def kernel(h_in, W_k, b_k, W_a1, b_a1, W_a2, b_a2, g_in, bt_in,
           g_pred, bt_pred, hash_proj, mem_vals, mem_counts):
    Bx, Lx, Dx = h_in.shape
    h2d = h_in.reshape(Bx * Lx, Dx)

    idx = _hash_call(h2d, W_k, hash_proj, b_k.reshape(1, Dx))

    pred, conf = _sc_gather_call(mem_vals, mem_counts, idx.reshape(NTOK))

    y, asum, csum = _gate_call(
        h2d, pred, conf.reshape(NTOK, 1), W_a1,
        b_a1.reshape(1, Dx), W_a2.reshape(1, Dx), b_a2.reshape(1, 1),
        g_in.reshape(1, Dx), bt_in.reshape(1, Dx),
        g_pred.reshape(1, Dx), bt_pred.reshape(1, Dx),
    )
    y_out = y.reshape(Bx, Lx, Dx)
    a_mean = jnp.sum(asum) / jnp.float32(NTOK)
    c_mean = jnp.sum(csum) / jnp.float32(NTOK)
    return (y_out, a_mean, c_mean)


# speedup vs baseline: 2.1745x; 1.0790x over previous
import jax
import jax.numpy as jnp
from jax import lax
from jax.experimental import pallas as pl
from jax.experimental.pallas import tpu as pltpu
from jax.experimental.pallas import tpu_sc as plsc

N_BITS = 14
N_SLOTS = 2 ** N_BITS
NTOK = 4096
D = 1024
BLK = 1024
GRID = NTOK // BLK

NC, NS, LANES = 2, 16, 16
NW = NC * NS
TPW = NTOK // NW
CH = 32
NCHUNK = TPW // CH


def _hash_body(wk_ref, hp_ref, bk_ref, h_ref, idx_ref, wh_s, bh_s):
    @pl.when(pl.program_id(0) == 0)
    def _fold():
        wh_s[...] = jnp.dot(wk_ref[...], hp_ref[...],
                            preferred_element_type=jnp.float32).astype(
                                jnp.bfloat16)
        bh_s[...] = jnp.dot(bk_ref[...], hp_ref[...],
                            preferred_element_type=jnp.float32)
    t = jnp.dot(h_ref[...].astype(jnp.bfloat16), wh_s[...],
                preferred_element_type=jnp.float32) + bh_s[...]
    lane = lax.broadcasted_iota(jnp.int32, (BLK, N_BITS), 1)
    powers = jnp.left_shift(1, lane)
    bits = (t > 0.0).astype(jnp.int32)
    idx_ref[...] = jnp.sum(bits * powers, axis=1).reshape(BLK // 128, 128)


def _hash_call(h2d, w_k, hp, bk2d):
    return pl.pallas_call(
        _hash_body,
        grid=(GRID,),
        in_specs=[
            pl.BlockSpec((1024, 1024), lambda i: (0, 0)),
            pl.BlockSpec((1024, N_BITS), lambda i: (0, 0)),
            pl.BlockSpec((1, 1024), lambda i: (0, 0)),
            pl.BlockSpec((BLK, 1024), lambda i: (i, 0)),
        ],
        out_specs=pl.BlockSpec((BLK // 128, 128), lambda i: (i, 0)),
        out_shape=jax.ShapeDtypeStruct((NTOK // 128, 128), jnp.int32),
        scratch_shapes=[
            pltpu.VMEM((1024, N_BITS), jnp.bfloat16),
            pltpu.VMEM((1, N_BITS), jnp.float32),
        ],
    )(w_k, hp, bk2d, h2d)


def _sc_gather_body(vals_hbm, counts_hbm, idx_hbm, pred_hbm, conf_hbm,
                    idx_v, counts_v, rows0, rows1, conf_v, sem0, sem1):
    wid = lax.axis_index("s") * NC + lax.axis_index("c")
    base = wid * TPW
    pltpu.sync_copy(idx_hbm.at[pl.ds(base, TPW)], idx_v)
    pltpu.sync_copy(counts_hbm, counts_v)
    for i in range(TPW // LANES):
        iv = idx_v[pl.ds(i * LANES, LANES)]
        c = plsc.load_gather(counts_v, [iv])
        conf_v[pl.ds(i * LANES, LANES)] = c / (c + 1.0)
    pltpu.sync_copy(conf_v, conf_hbm.at[pl.ds(base, TPW)])

    rows = (rows0, rows1)
    sems = (sem0, sem1)
    cps = []
    for k in range(NCHUNK):
        b = k % 2
        if k >= 2:
            cps[k - 2].wait()
            pltpu.sync_copy(rows[b],
                            pred_hbm.at[pl.ds(base + (k - 2) * CH, CH)])
        cp = pltpu.async_copy(vals_hbm.at[idx_v.at[pl.ds(k * CH, CH)]],
                              rows[b], sems[b])
        cps.append(cp)
    for k in range(max(NCHUNK - 2, 0), NCHUNK):
        cps[k].wait()
        pltpu.sync_copy(rows[k % 2], pred_hbm.at[pl.ds(base + k * CH, CH)])


def _sc_gather_call(mem_vals, mem_counts, idx1d):
    mesh = plsc.VectorSubcoreMesh(core_axis_name="c", subcore_axis_name="s",
                                  num_cores=NC, num_subcores=NS)
    fn = pl.kernel(
        _sc_gather_body,
        compiler_params=pltpu.CompilerParams(needs_layout_passes=False),
        out_type=[
            jax.ShapeDtypeStruct((NTOK, D), jnp.float32),
            jax.ShapeDtypeStruct((NTOK,), jnp.float32),
        ],
        mesh=mesh,
        scratch_types=[
            pltpu.VMEM((TPW,), jnp.int32),
            pltpu.VMEM((N_SLOTS,), jnp.float32),
            pltpu.VMEM((CH, D), jnp.float32),
            pltpu.VMEM((CH, D), jnp.float32),
            pltpu.VMEM((TPW,), jnp.float32),
            pltpu.SemaphoreType.DMA,
            pltpu.SemaphoreType.DMA,
        ],
    )
    return fn(mem_vals, mem_counts, idx1d)


def _ln(x, g, b):
    mu = jnp.mean(x, axis=1, keepdims=True)
    xc = x - mu
    var = jnp.mean(xc * xc, axis=1, keepdims=True)
    return xc * lax.rsqrt(var + 1e-5) * g + b


def _gate_body(h_ref, pred_ref, conf_ref, w1h_ref, w1p_ref, w1c_ref, ba1_ref,
               wa2_ref, ba2_ref, gin_ref, btin_ref, gp_ref, btp_ref,
               y_ref, asum_ref, csum_ref, w1hb_s, w1pb_s):
    @pl.when(pl.program_id(0) == 0)
    def _cast():
        w1hb_s[...] = w1h_ref[...].astype(jnp.bfloat16)
        w1pb_s[...] = w1p_ref[...].astype(jnp.bfloat16)
    x = h_ref[...]
    p = pred_ref[...]
    ct = conf_ref[...]
    tok_r = lax.broadcasted_iota(jnp.int32, (BLK, 8), 0) // 128
    sel = (tok_r == lax.broadcasted_iota(jnp.int32, (BLK, 8), 1)
           ).astype(jnp.float32)
    cf_rows = jnp.dot(sel, ct, preferred_element_type=jnp.float32)
    tok_l = lax.broadcasted_iota(jnp.int32, (BLK, 128), 0) % 128
    lmask = tok_l == lax.broadcasted_iota(jnp.int32, (BLK, 128), 1)
    cf = jnp.sum(jnp.where(lmask, cf_rows, 0.0), axis=1, keepdims=True)
    ln_h = _ln(x, gin_ref[...], btin_ref[...])
    ln_p = _ln(p, gp_ref[...], btp_ref[...])
    z = (jnp.dot(ln_h.astype(jnp.bfloat16), w1hb_s[...],
                 preferred_element_type=jnp.float32)
         + jnp.dot(ln_p.astype(jnp.bfloat16), w1pb_s[...],
                   preferred_element_type=jnp.float32)
         + cf * w1c_ref[0:1, :] + ba1_ref[...])
    hid = z * jax.nn.sigmoid(z)
    s = jnp.dot(hid.astype(jnp.bfloat16), wa2_ref[...].astype(jnp.bfloat16),
                preferred_element_type=jnp.float32) + ba2_ref[...]
    alpha = jax.nn.sigmoid(s)
    y_ref[...] = x + alpha * p

    @pl.when(pl.program_id(0) == 0)
    def _init():
        asum_ref[0, 0] = 0.0
        csum_ref[0, 0] = 0.0
    asum_ref[0, 0] += jnp.sum(alpha)
    csum_ref[0, 0] += jnp.sum(cf)


def _gate_call(h2d, pred, conf2d, w_a1, ba1, wa2, ba2,
               gin, btin, gp, btp):
    def full(r, c):
        return pl.BlockSpec((r, c), lambda i: (0, 0))
    return pl.pallas_call(
        _gate_body,
        grid=(GRID,),
        in_specs=[
            pl.BlockSpec((BLK, D), lambda i: (i, 0)),
            pl.BlockSpec((BLK, D), lambda i: (i, 0)),
            pl.BlockSpec((BLK // 128, 128), lambda i: (i, 0)),
            pl.BlockSpec((1024, 1024), lambda i: (0, 0)),
            pl.BlockSpec((1024, 1024), lambda i: (1, 0)),
            pl.BlockSpec((8, 1024), lambda i: (256, 0)),
            full(1, 1024),
            full(1024, 1),
            full(1, 1),
            full(1, 1024),
            full(1, 1024),
            full(1, 1024),
            full(1, 1024),
        ],
        out_specs=[
            pl.BlockSpec((BLK, D), lambda i: (i, 0)),
            pl.BlockSpec((1, 1), lambda i: (0, 0),
                         memory_space=pltpu.SMEM),
            pl.BlockSpec((1, 1), lambda i: (0, 0),
                         memory_space=pltpu.SMEM),
        ],
        out_shape=[
            jax.ShapeDtypeStruct((NTOK, D), jnp.float32),
            jax.ShapeDtypeStruct((1, 1), jnp.float32),
            jax.ShapeDtypeStruct((1, 1), jnp.float32),
        ],
        scratch_shapes=[
            pltpu.VMEM((1024, 1024), jnp.bfloat16),
            pltpu.VMEM((1024, 1024), jnp.bfloat16),
        ],
    )(h2d, pred, conf2d, w_a1, w_a1, w_a1, ba1, wa2, ba2, gin, btin, gp, btp)


def kernel(h_in, W_k, b_k, W_a1, b_a1, W_a2, b_a2, g_in, bt_in,
           g_pred, bt_pred, hash_proj, mem_vals, mem_counts):
    Bx, Lx, Dx = h_in.shape
    h2d = h_in.reshape(Bx * Lx, Dx)

    idx = _hash_call(h2d, W_k, hash_proj, b_k.reshape(1, Dx))

    pred, conf = _sc_gather_call(mem_vals, mem_counts, idx.reshape(NTOK))

    y, asum, csum = _gate_call(
        h2d, pred, conf.reshape(NTOK // 128, 128), W_a1,
        b_a1.reshape(1, Dx), W_a2, b_a2.reshape(1, 1),
        g_in.reshape(1, Dx), bt_in.reshape(1, Dx),
        g_pred.reshape(1, Dx), bt_pred.reshape(1, Dx),
    )
    y_out = y.reshape(Bx, Lx, Dx)
    a_mean = jnp.sum(asum) / jnp.float32(NTOK)
    c_mean = jnp.sum(csum) / jnp.float32(NTOK)
    return (y_out, a_mean, c_mean)
